# Initial kernel scaffold; baseline (speedup 1.0000x reference)
import jax
import jax.numpy as jnp
from jax.experimental import pallas as pl


def kernel(h, x, edge_index, edge_attr, W_in, b_in, We1, be1, We2, be2, Wc1, bc1, Wc2, bc2, Wn1, bn1, Wn2, bn2, W_out, b_out):
    raise NotImplementedError("write your pallas kernel here")


# speedup vs baseline: 2.8446x; 2.8446x over previous
import functools

import jax
import jax.numpy as jnp
from jax import lax
from jax.experimental import pallas as pl
from jax.experimental.pallas import tpu as pltpu
from jax.experimental.pallas import tpu_sc as plsc

F32 = jnp.float32
NW = 32
CHUNK = 128
PW = 72


def _silu(v):
    return v * jax.nn.sigmoid(v)


def _pre_body(h_r, Win_r, bin_r, We1a_r, We1b_r, be1_r, hh_r, A_r, B_r):
    hh = jnp.dot(h_r[...], Win_r[...], preferred_element_type=F32) + bin_r[...]
    hh_r[...] = hh
    A_r[...] = jnp.dot(hh, We1a_r[...], preferred_element_type=F32) + be1_r[...]
    B_r[...] = jnp.dot(hh, We1b_r[...], preferred_element_type=F32)


def _tc_pre(h, W_in, b_in, We1a, We1b, be1, bn):
    n, in_nf = h.shape
    hf = W_in.shape[1]
    grid = (n // bn,)
    full = lambda a: pl.BlockSpec(a.shape, lambda i: (0,) * a.ndim)
    return pl.pallas_call(
        _pre_body,
        grid=grid,
        in_specs=[pl.BlockSpec((bn, in_nf), lambda i: (i, 0)),
                  full(W_in), full(b_in), full(We1a), full(We1b), full(be1)],
        out_specs=[pl.BlockSpec((bn, hf), lambda i: (i, 0))] * 3,
        out_shape=[jax.ShapeDtypeStruct((n, hf), F32)] * 3,
    )(h, W_in, b_in, We1a, We1b, be1)


def _sc_gather(A, B, x4, rowg2d, colg2d, e_pad, cpw):
    hf = A.shape[1]
    mesh = plsc.VectorSubcoreMesh(core_axis_name="c", subcore_axis_name="s")

    @functools.partial(
        pl.kernel,
        mesh=mesh,
        out_type=(jax.ShapeDtypeStruct((e_pad, hf), F32),
                  jax.ShapeDtypeStruct((e_pad, hf), F32),
                  jax.ShapeDtypeStruct((e_pad, 8), F32),
                  jax.ShapeDtypeStruct((e_pad, 8), F32)),
        scratch_types=[pltpu.VMEM((cpw, CHUNK), jnp.int32),
                       pltpu.VMEM((cpw, CHUNK), jnp.int32),
                       pltpu.VMEM((CHUNK, hf), F32),
                       pltpu.VMEM((CHUNK, hf), F32),
                       pltpu.VMEM((CHUNK, 8), F32),
                       pltpu.VMEM((CHUNK, 8), F32),
                       pltpu.SemaphoreType.DMA],
        compiler_params=pltpu.CompilerParams(use_tc_tiling_on_sc=False),
    )
    def k(A_h, B_h, x_h, rg_h, cg_h, oA, oB, oxr, oxc,
          rowv, colv, bufA, bufB, bufxr, bufxc, sem):
        c = lax.axis_index("c")
        s = lax.axis_index("s")
        wid = c * 16 + s
        base_chunk = wid * cpw
        pltpu.sync_copy(rg_h.at[wid], rowv)
        pltpu.sync_copy(cg_h.at[wid], colv)

        def body(j, carry):
            off = (base_chunk + j) * CHUNK
            pltpu.async_copy(A_h.at[rowv.at[j]], bufA, sem).wait()
            pltpu.sync_copy(bufA, oA.at[pl.ds(off, CHUNK)])
            pltpu.async_copy(B_h.at[colv.at[j]], bufB, sem).wait()
            pltpu.sync_copy(bufB, oB.at[pl.ds(off, CHUNK)])
            pltpu.async_copy(x_h.at[rowv.at[j]], bufxr, sem).wait()
            pltpu.sync_copy(bufxr, oxr.at[pl.ds(off, CHUNK)])
            pltpu.async_copy(x_h.at[colv.at[j]], bufxc, sem).wait()
            pltpu.sync_copy(bufxc, oxc.at[pl.ds(off, CHUNK)])
            return carry

        lax.fori_loop(0, cpw, body, 0)

    return k(A, B, x4, rowg2d, colg2d)


def _edge_body(hhA_r, hhB_r, xr_r, xc_r, ea_r, We1c_r, wr_r, We2_r, be2_r,
               Wc1_r, bc1_r, Wc2_r, bc2_r, out_r):
    d = xr_r[...] - xc_r[...]
    radial = jnp.sum(d * d, axis=1, keepdims=True)
    pre1 = (hhA_r[...] + hhB_r[...] + radial * wr_r[...]
            + jnp.dot(ea_r[...], We1c_r[...], preferred_element_type=F32))
    m = _silu(pre1)
    m = _silu(jnp.dot(m, We2_r[...], preferred_element_type=F32) + be2_r[...])
    p = _silu(jnp.dot(m, Wc1_r[...], preferred_element_type=F32) + bc1_r[...])
    cw = jnp.dot(p, Wc2_r[...], preferred_element_type=F32) + bc2_r[...]
    trans = (d * cw)[:, 0:4]
    be = m.shape[0]
    ones = jnp.ones((be, 1), F32)
    zeros = jnp.zeros((be, PW - 69), F32)
    out_r[...] = jnp.concatenate([m, trans, ones, zeros], axis=1)


def _tc_edge(hhA, hhB, xr4, xc4, ea, We1c, wr, We2, be2, Wc1, bc1, Wc2, bc2, be):
    e_pad, hf = hhA.shape
    enf = ea.shape[1]
    grid = (e_pad // be,)
    full = lambda a: pl.BlockSpec(a.shape, lambda i: (0,) * a.ndim)
    return pl.pallas_call(
        _edge_body,
        grid=grid,
        in_specs=[pl.BlockSpec((be, hf), lambda i: (i, 0)),
                  pl.BlockSpec((be, hf), lambda i: (i, 0)),
                  pl.BlockSpec((be, 8), lambda i: (i, 0)),
                  pl.BlockSpec((be, 8), lambda i: (i, 0)),
                  pl.BlockSpec((be, enf), lambda i: (i, 0)),
                  full(We1c), full(wr), full(We2), full(be2),
                  full(Wc1), full(bc1), full(Wc2), full(bc2)],
        out_specs=pl.BlockSpec((be, PW), lambda i: (i, 0)),
        out_shape=jax.ShapeDtypeStruct((e_pad, PW), F32),
    )(hhA, hhB, xr4, xc4, ea, We1c, wr, We2, be2, Wc1, bc1, Wc2, bc2)


def _sc_scatter(comb, rs2d, zeros, n_acc, cpw):
    rows_per_tile = n_acc // 16
    mesh = plsc.VectorSubcoreMesh(core_axis_name="c", subcore_axis_name="s")

    @functools.partial(
        pl.kernel,
        mesh=mesh,
        out_type=jax.ShapeDtypeStruct((2, n_acc, PW), F32),
        scratch_types=[pltpu.VMEM((cpw, CHUNK), jnp.int32),
                       pltpu.VMEM((CHUNK, PW), F32),
                       pltpu.VMEM_SHARED((n_acc, PW), F32)],
        compiler_params=pltpu.CompilerParams(use_tc_tiling_on_sc=False),
    )
    def k(comb_h, rs_h, z_h, out_h, rsv, buf, acc):
        c = lax.axis_index("c")
        s = lax.axis_index("s")
        wid = c * 16 + s
        pltpu.sync_copy(z_h.at[pl.ds(s * rows_per_tile, rows_per_tile)],
                        acc.at[pl.ds(s * rows_per_tile, rows_per_tile)])
        pltpu.sync_copy(rs_h.at[wid], rsv)
        plsc.subcore_barrier()

        def body(j, carry):
            off = (wid * cpw + j) * CHUNK
            pltpu.sync_copy(comb_h.at[pl.ds(off, CHUNK)], buf)
            pltpu.sync_copy(buf, acc.at[rsv.at[j]], add=True)
            return carry

        lax.fori_loop(0, cpw, body, 0)
        plsc.subcore_barrier()
        pltpu.sync_copy(acc.at[pl.ds(s * rows_per_tile, rows_per_tile)],
                        out_h.at[c, pl.ds(s * rows_per_tile, rows_per_tile)])

    return k(comb, rs2d, zeros)


def _node_body(hh_r, x_r, p0_r, p1_r, Wn1a_r, Wn1b_r, bn1_r, Wn2_r, bn2_r,
               Wo_r, bo_r, out_r):
    p0 = p0_r[...]
    p1 = p1_r[...]
    magg = p0[:, 0:64] + p1[:, 0:64]
    tsum = p0[:, 64:67] + p1[:, 64:67]
    cnt = p0[:, 68:69] + p1[:, 68:69]
    x_out = x_r[...] + tsum / jnp.maximum(cnt, 1.0)
    hh = hh_r[...]
    h2 = _silu(jnp.dot(hh, Wn1a_r[...], preferred_element_type=F32)
               + jnp.dot(magg, Wn1b_r[...], preferred_element_type=F32)
               + bn1_r[...])
    h2 = jnp.dot(h2, Wn2_r[...], preferred_element_type=F32) + bn2_r[...]
    emb = jnp.dot(h2, Wo_r[...], preferred_element_type=F32) + bo_r[...]
    out_r[...] = jnp.concatenate([emb, x_out], axis=1)


def _tc_node(hh, x, p0, p1, Wn1a, Wn1b, bn1, Wn2, bn2, W_out, b_out, bn):
    n, hf = hh.shape
    emb_nf = W_out.shape[1]
    grid = (n // bn,)
    full = lambda a: pl.BlockSpec(a.shape, lambda i: (0,) * a.ndim)
    return pl.pallas_call(
        _node_body,
        grid=grid,
        in_specs=[pl.BlockSpec((bn, hf), lambda i: (i, 0)),
                  pl.BlockSpec((bn, 3), lambda i: (i, 0)),
                  pl.BlockSpec((bn, PW), lambda i: (i, 0)),
                  pl.BlockSpec((bn, PW), lambda i: (i, 0)),
                  full(Wn1a), full(Wn1b), full(bn1), full(Wn2), full(bn2),
                  full(W_out), full(b_out)],
        out_specs=pl.BlockSpec((bn, emb_nf + 3), lambda i: (i, 0)),
        out_shape=jax.ShapeDtypeStruct((n, emb_nf + 3), F32),
    )(hh, x, p0, p1, Wn1a, Wn1b, bn1, Wn2, bn2, W_out, b_out)


def kernel(h, x, edge_index, edge_attr, W_in, b_in, We1, be1, We2, be2,
           Wc1, bc1, Wc2, bc2, Wn1, bn1, Wn2, bn2, W_out, b_out):
    n = h.shape[0]
    e = edge_index.shape[1]
    hf = W_in.shape[1]

    cpw = -(-e // (NW * CHUNK))
    e_pad = NW * cpw * CHUNK
    pad = e_pad - e
    n_acc = ((n + 1 + 127) // 128) * 128

    row = edge_index[0].astype(jnp.int32)
    col = edge_index[1].astype(jnp.int32)
    rowg2d = jnp.concatenate([row, jnp.zeros((pad,), jnp.int32)]).reshape(NW, cpw, CHUNK)
    colg2d = jnp.concatenate([col, jnp.zeros((pad,), jnp.int32)]).reshape(NW, cpw, CHUNK)
    rs2d = jnp.concatenate([row, jnp.full((pad,), n, jnp.int32)]).reshape(NW, cpw, CHUNK)
    ea_pad = jnp.pad(edge_attr, ((0, pad), (0, 0)))
    x4 = jnp.pad(x, ((0, 0), (0, 5)))

    We1a = We1[:hf]
    We1b = We1[hf:2 * hf]
    wr = We1[2 * hf:2 * hf + 1]
    We1c = We1[2 * hf + 1:]
    b2 = lambda v: v.reshape(1, -1)

    hh, A, B = _tc_pre(h, W_in, b2(b_in), We1a, We1b, b2(be1), bn=1000)
    hhA, hhB, xr4, xc4 = _sc_gather(A, B, x4, rowg2d, colg2d, e_pad, cpw)
    comb = _tc_edge(hhA, hhB, xr4, xc4, ea_pad, We1c, wr, We2, b2(be2),
                    Wc1, b2(bc1), Wc2, b2(bc2), be=2048)
    zeros = jnp.zeros((n_acc, PW), F32)
    partials = _sc_scatter(comb, rs2d, zeros, n_acc, cpw)
    p0 = partials[0, :n]
    p1 = partials[1, :n]
    out = _tc_node(hh, x, p0, p1, Wn1[:hf], Wn1[hf:], b2(bn1), Wn2, b2(bn2),
                   W_out, b2(b_out), bn=1000)
    return out


# speedup vs baseline: 3.4511x; 1.2132x over previous
import functools

import jax
import jax.numpy as jnp
from jax import lax
from jax.experimental import pallas as pl
from jax.experimental.pallas import tpu as pltpu
from jax.experimental.pallas import tpu_sc as plsc

F32 = jnp.float32
NW = 32
CHUNK = 128
PW = 72


def _silu(v):
    return v * jax.nn.sigmoid(v)


def _pre_body(h_r, Win_r, bin_r, We1a_r, We1b_r, be1_r, hh_r, A_r, B_r):
    hh = jnp.dot(h_r[...], Win_r[...], preferred_element_type=F32) + bin_r[...]
    hh_r[...] = hh
    A_r[...] = jnp.dot(hh, We1a_r[...], preferred_element_type=F32) + be1_r[...]
    B_r[...] = jnp.dot(hh, We1b_r[...], preferred_element_type=F32)


def _tc_pre(h, W_in, b_in, We1a, We1b, be1, bn):
    n, in_nf = h.shape
    hf = W_in.shape[1]
    grid = (n // bn,)
    full = lambda a: pl.BlockSpec(a.shape, lambda i: (0,) * a.ndim)
    return pl.pallas_call(
        _pre_body,
        grid=grid,
        in_specs=[pl.BlockSpec((bn, in_nf), lambda i: (i, 0)),
                  full(W_in), full(b_in), full(We1a), full(We1b), full(be1)],
        out_specs=[pl.BlockSpec((bn, hf), lambda i: (i, 0))] * 3,
        out_shape=[jax.ShapeDtypeStruct((n, hf), F32)] * 3,
    )(h, W_in, b_in, We1a, We1b, be1)


def _sc_gather(A, B, x4, rowg2d, colg2d, e_pad, cpw):
    hf = A.shape[1]
    mesh = plsc.VectorSubcoreMesh(core_axis_name="c", subcore_axis_name="s")

    @functools.partial(
        pl.kernel,
        mesh=mesh,
        out_type=(jax.ShapeDtypeStruct((e_pad, hf), F32),
                  jax.ShapeDtypeStruct((e_pad, 8), F32),
                  jax.ShapeDtypeStruct((e_pad, 8), F32)),
        scratch_types=[pltpu.VMEM((cpw, CHUNK), jnp.int32),
                       pltpu.VMEM((cpw, CHUNK), jnp.int32),
                       pltpu.VMEM((2, CHUNK, hf), F32),
                       pltpu.VMEM((2, CHUNK, hf), F32),
                       pltpu.VMEM((2, CHUNK, 8), F32),
                       pltpu.VMEM((2, CHUNK, 8), F32),
                       pltpu.SemaphoreType.DMA,
                       pltpu.SemaphoreType.DMA,
                       pltpu.SemaphoreType.DMA,
                       pltpu.SemaphoreType.DMA],
        compiler_params=pltpu.CompilerParams(use_tc_tiling_on_sc=False),
    )
    def k(A_h, B_h, x_h, rg_h, cg_h, oT, oxr, oxc,
          rowv, colv, bufA, bufB, bufxr, bufxc, g0, g1, w0, w1):
        c = lax.axis_index("c")
        s = lax.axis_index("s")
        wid = c * 16 + s
        base_chunk = wid * cpw
        pltpu.sync_copy(rg_h.at[wid], rowv)
        pltpu.sync_copy(cg_h.at[wid], colv)
        gs = (g0, g1)
        ws = (w0, w1)

        def issue_g(j, b):
            pltpu.async_copy(A_h.at[rowv.at[j]], bufA.at[b], gs[b])
            pltpu.async_copy(B_h.at[colv.at[j]], bufB.at[b], gs[b])
            pltpu.async_copy(x_h.at[rowv.at[j]], bufxr.at[b], gs[b])
            pltpu.async_copy(x_h.at[colv.at[j]], bufxc.at[b], gs[b])

        def drain_g(j, b):
            pltpu.make_async_copy(A_h.at[rowv.at[j]], bufA.at[b], gs[b]).wait()
            pltpu.make_async_copy(B_h.at[colv.at[j]], bufB.at[b], gs[b]).wait()
            pltpu.make_async_copy(x_h.at[rowv.at[j]], bufxr.at[b], gs[b]).wait()
            pltpu.make_async_copy(x_h.at[colv.at[j]], bufxc.at[b], gs[b]).wait()

        def issue_w(j, b):
            off = (base_chunk + j) * CHUNK
            pltpu.async_copy(bufA.at[b], oT.at[pl.ds(off, CHUNK)], ws[b])
            pltpu.async_copy(bufxr.at[b], oxr.at[pl.ds(off, CHUNK)], ws[b])
            pltpu.async_copy(bufxc.at[b], oxc.at[pl.ds(off, CHUNK)], ws[b])

        def drain_w(j, b):
            off = (base_chunk + j) * CHUNK
            pltpu.make_async_copy(bufA.at[b], oT.at[pl.ds(off, CHUNK)], ws[b]).wait()
            pltpu.make_async_copy(bufxr.at[b], oxr.at[pl.ds(off, CHUNK)], ws[b]).wait()
            pltpu.make_async_copy(bufxc.at[b], oxc.at[pl.ds(off, CHUNK)], ws[b]).wait()

        issue_g(0, 0)

        def outer(jj, carry):
            for b in (0, 1):
                j = jj * 2 + b
                drain_g(j, b)

                def add_body(i, carry2):
                    for cc in range(hf // 16):
                        sl = pl.ds(cc * 16, 16)
                        bufA[b, i, sl] = bufA[b, i, sl] + bufB[b, i, sl]
                    return carry2

                lax.fori_loop(0, CHUNK, add_body, 0)

                @pl.when(j >= 1)
                def _():
                    drain_w(j - 1, 1 - b)

                @pl.when(j + 1 < cpw)
                def _():
                    issue_g(j + 1, 1 - b)

                issue_w(j, b)
            return carry

        lax.fori_loop(0, cpw // 2, outer, 0)
        drain_w(cpw - 1, 1)

    return k(A, B, x4, rowg2d, colg2d)


def _edge_body(t1_r, xr_r, xc_r, ea_r, We1c_r, wr_r, We2_r, be2_r,
               Wc1_r, bc1_r, Wc2_r, bc2_r, out_r):
    d = xr_r[...] - xc_r[...]
    radial = jnp.sum(d * d, axis=1, keepdims=True)
    pre1 = (t1_r[...] + radial * wr_r[...]
            + jnp.dot(ea_r[...], We1c_r[...], preferred_element_type=F32))
    m = _silu(pre1)
    m = _silu(jnp.dot(m, We2_r[...], preferred_element_type=F32) + be2_r[...])
    p = _silu(jnp.dot(m, Wc1_r[...], preferred_element_type=F32) + bc1_r[...])
    cw = jnp.dot(p, Wc2_r[...], preferred_element_type=F32) + bc2_r[...]
    trans = (d * cw)[:, 0:4]
    be = m.shape[0]
    ones = jnp.ones((be, 1), F32)
    zeros = jnp.zeros((be, PW - 69), F32)
    out_r[...] = jnp.concatenate([m, trans, ones, zeros], axis=1)


def _tc_edge(t1, xr4, xc4, ea, We1c, wr, We2, be2, Wc1, bc1, Wc2, bc2, be):
    e_pad, hf = t1.shape
    enf = ea.shape[1]
    grid = (e_pad // be,)
    full = lambda a: pl.BlockSpec(a.shape, lambda i: (0,) * a.ndim)
    return pl.pallas_call(
        _edge_body,
        grid=grid,
        in_specs=[pl.BlockSpec((be, hf), lambda i: (i, 0)),
                  pl.BlockSpec((be, 8), lambda i: (i, 0)),
                  pl.BlockSpec((be, 8), lambda i: (i, 0)),
                  pl.BlockSpec((be, enf), lambda i: (i, 0)),
                  full(We1c), full(wr), full(We2), full(be2),
                  full(Wc1), full(bc1), full(Wc2), full(bc2)],
        out_specs=pl.BlockSpec((be, PW), lambda i: (i, 0)),
        out_shape=jax.ShapeDtypeStruct((e_pad, PW), F32),
    )(t1, xr4, xc4, ea, We1c, wr, We2, be2, Wc1, bc1, Wc2, bc2)


def _sc_scatter(comb, rs2d, zeros, n_acc, cpw):
    rows_per_tile = n_acc // 16
    mesh = plsc.VectorSubcoreMesh(core_axis_name="c", subcore_axis_name="s")

    @functools.partial(
        pl.kernel,
        mesh=mesh,
        out_type=jax.ShapeDtypeStruct((2, n_acc, PW), F32),
        scratch_types=[pltpu.VMEM((cpw, CHUNK), jnp.int32),
                       pltpu.VMEM((2, CHUNK, PW), F32),
                       pltpu.VMEM_SHARED((n_acc, PW), F32),
                       pltpu.SemaphoreType.DMA,
                       pltpu.SemaphoreType.DMA],
        compiler_params=pltpu.CompilerParams(use_tc_tiling_on_sc=False),
    )
    def k(comb_h, rs_h, z_h, out_h, rsv, buf, acc, l0, l1):
        c = lax.axis_index("c")
        s = lax.axis_index("s")
        wid = c * 16 + s
        pltpu.sync_copy(z_h.at[pl.ds(s * rows_per_tile, rows_per_tile)],
                        acc.at[pl.ds(s * rows_per_tile, rows_per_tile)])
        pltpu.sync_copy(rs_h.at[wid], rsv)
        plsc.subcore_barrier()
        ls = (l0, l1)

        def issue_l(j, b):
            off = (wid * cpw + j) * CHUNK
            pltpu.async_copy(comb_h.at[pl.ds(off, CHUNK)], buf.at[b], ls[b])

        def drain_l(j, b):
            off = (wid * cpw + j) * CHUNK
            pltpu.make_async_copy(comb_h.at[pl.ds(off, CHUNK)], buf.at[b],
                                  ls[b]).wait()

        issue_l(0, 0)

        def body(jj, carry):
            for b in (0, 1):
                j = jj * 2 + b
                drain_l(j, b)

                @pl.when(j + 1 < cpw)
                def _():
                    issue_l(j + 1, 1 - b)

                pltpu.sync_copy(buf.at[b], acc.at[rsv.at[j]], add=True)
            return carry

        lax.fori_loop(0, cpw // 2, body, 0)
        plsc.subcore_barrier()
        pltpu.sync_copy(acc.at[pl.ds(s * rows_per_tile, rows_per_tile)],
                        out_h.at[c, pl.ds(s * rows_per_tile, rows_per_tile)])

    return k(comb, rs2d, zeros)


def _node_body(hh_r, x_r, p0_r, p1_r, Wn1a_r, Wn1b_r, bn1_r, Wn2_r, bn2_r,
               Wo_r, bo_r, out_r):
    p0 = p0_r[...]
    p1 = p1_r[...]
    magg = p0[:, 0:64] + p1[:, 0:64]
    tsum = p0[:, 64:67] + p1[:, 64:67]
    cnt = p0[:, 68:69] + p1[:, 68:69]
    x_out = x_r[...] + tsum / jnp.maximum(cnt, 1.0)
    hh = hh_r[...]
    h2 = _silu(jnp.dot(hh, Wn1a_r[...], preferred_element_type=F32)
               + jnp.dot(magg, Wn1b_r[...], preferred_element_type=F32)
               + bn1_r[...])
    h2 = jnp.dot(h2, Wn2_r[...], preferred_element_type=F32) + bn2_r[...]
    emb = jnp.dot(h2, Wo_r[...], preferred_element_type=F32) + bo_r[...]
    out_r[...] = jnp.concatenate([emb, x_out], axis=1)


def _tc_node(hh, x, p0, p1, Wn1a, Wn1b, bn1, Wn2, bn2, W_out, b_out, bn):
    n, hf = hh.shape
    emb_nf = W_out.shape[1]
    grid = (n // bn,)
    full = lambda a: pl.BlockSpec(a.shape, lambda i: (0,) * a.ndim)
    return pl.pallas_call(
        _node_body,
        grid=grid,
        in_specs=[pl.BlockSpec((bn, hf), lambda i: (i, 0)),
                  pl.BlockSpec((bn, 3), lambda i: (i, 0)),
                  pl.BlockSpec((bn, PW), lambda i: (i, 0)),
                  pl.BlockSpec((bn, PW), lambda i: (i, 0)),
                  full(Wn1a), full(Wn1b), full(bn1), full(Wn2), full(bn2),
                  full(W_out), full(b_out)],
        out_specs=pl.BlockSpec((bn, emb_nf + 3), lambda i: (i, 0)),
        out_shape=jax.ShapeDtypeStruct((n, emb_nf + 3), F32),
    )(hh, x, p0, p1, Wn1a, Wn1b, bn1, Wn2, bn2, W_out, b_out)


def kernel(h, x, edge_index, edge_attr, W_in, b_in, We1, be1, We2, be2,
           Wc1, bc1, Wc2, bc2, Wn1, bn1, Wn2, bn2, W_out, b_out):
    n = h.shape[0]
    e = edge_index.shape[1]
    hf = W_in.shape[1]

    cpw = -(-e // (NW * CHUNK))
    cpw = cpw + (cpw % 2)
    e_pad = NW * cpw * CHUNK
    pad = e_pad - e
    n_acc = ((n + 1 + 127) // 128) * 128

    row = edge_index[0].astype(jnp.int32)
    col = edge_index[1].astype(jnp.int32)
    rowg2d = jnp.concatenate([row, jnp.zeros((pad,), jnp.int32)]).reshape(NW, cpw, CHUNK)
    colg2d = jnp.concatenate([col, jnp.zeros((pad,), jnp.int32)]).reshape(NW, cpw, CHUNK)
    rs2d = jnp.concatenate([row, jnp.full((pad,), n, jnp.int32)]).reshape(NW, cpw, CHUNK)
    ea_pad = jnp.pad(edge_attr, ((0, pad), (0, 0)))
    x4 = jnp.pad(x, ((0, 0), (0, 5)))

    We1a = We1[:hf]
    We1b = We1[hf:2 * hf]
    wr = We1[2 * hf:2 * hf + 1]
    We1c = We1[2 * hf + 1:]
    b2 = lambda v: v.reshape(1, -1)

    hh, A, B = _tc_pre(h, W_in, b2(b_in), We1a, We1b, b2(be1), bn=1000)
    t1, xr4, xc4 = _sc_gather(A, B, x4, rowg2d, colg2d, e_pad, cpw)
    comb = _tc_edge(t1, xr4, xc4, ea_pad, We1c, wr, We2, b2(be2),
                    Wc1, b2(bc1), Wc2, b2(bc2), be=2048)
    zeros = jnp.zeros((n_acc, PW), F32)
    partials = _sc_scatter(comb, rs2d, zeros, n_acc, cpw)
    p0 = partials[0, :n]
    p1 = partials[1, :n]
    out = _tc_node(hh, x, p0, p1, Wn1[:hf], Wn1[hf:], b2(bn1), Wn2, b2(bn2),
                   W_out, b2(b_out), bn=1000)
    return out


# speedup vs baseline: 3.4986x; 1.0138x over previous
import functools

import jax
import jax.numpy as jnp
from jax import lax
from jax.experimental import pallas as pl
from jax.experimental.pallas import tpu as pltpu
from jax.experimental.pallas import tpu_sc as plsc

F32 = jnp.float32
NW = 32
CHUNK = 128
PW = 72


def _silu(v):
    return v * jax.nn.sigmoid(v)


def _pre_body(h_r, Win_r, bin_r, We1a_r, We1b_r, be1_r, hh_r, A_r, B_r):
    hh = jnp.dot(h_r[...], Win_r[...], preferred_element_type=F32) + bin_r[...]
    hh_r[...] = hh
    A_r[...] = jnp.dot(hh, We1a_r[...], preferred_element_type=F32) + be1_r[...]
    B_r[...] = jnp.dot(hh, We1b_r[...], preferred_element_type=F32)


def _tc_pre(h, W_in, b_in, We1a, We1b, be1, bn):
    n, in_nf = h.shape
    hf = W_in.shape[1]
    grid = (n // bn,)
    full = lambda a: pl.BlockSpec(a.shape, lambda i: (0,) * a.ndim)
    return pl.pallas_call(
        _pre_body,
        grid=grid,
        in_specs=[pl.BlockSpec((bn, in_nf), lambda i: (i, 0)),
                  full(W_in), full(b_in), full(We1a), full(We1b), full(be1)],
        out_specs=[pl.BlockSpec((bn, hf), lambda i: (i, 0))] * 3,
        out_shape=[jax.ShapeDtypeStruct((n, hf), F32)] * 3,
    )(h, W_in, b_in, We1a, We1b, be1)


def _sc_gather(A, B, x4, rowg2d, colg2d, e_pad, cpw):
    hf = A.shape[1]
    mesh = plsc.VectorSubcoreMesh(core_axis_name="c", subcore_axis_name="s")

    @functools.partial(
        pl.kernel,
        mesh=mesh,
        out_type=(jax.ShapeDtypeStruct((e_pad, hf), F32),
                  jax.ShapeDtypeStruct((e_pad, 8), F32),
                  jax.ShapeDtypeStruct((e_pad, 8), F32)),
        scratch_types=[pltpu.VMEM((cpw, CHUNK), jnp.int32),
                       pltpu.VMEM((cpw, CHUNK), jnp.int32),
                       pltpu.VMEM((2, CHUNK, hf), F32),
                       pltpu.VMEM((2, CHUNK, hf), F32),
                       pltpu.VMEM((2, CHUNK, 8), F32),
                       pltpu.VMEM((2, CHUNK, 8), F32),
                       pltpu.SemaphoreType.DMA,
                       pltpu.SemaphoreType.DMA,
                       pltpu.SemaphoreType.DMA,
                       pltpu.SemaphoreType.DMA],
        compiler_params=pltpu.CompilerParams(use_tc_tiling_on_sc=False),
    )
    def k(A_h, B_h, x_h, rg_h, cg_h, oT, oxr, oxc,
          rowv, colv, bufA, bufB, bufxr, bufxc, g0, g1, w0, w1):
        c = lax.axis_index("c")
        s = lax.axis_index("s")
        wid = c * 16 + s
        base_chunk = wid * cpw
        pltpu.sync_copy(rg_h.at[wid], rowv)
        pltpu.sync_copy(cg_h.at[wid], colv)
        gs = (g0, g1)
        ws = (w0, w1)

        def issue_g(j, b):
            pltpu.async_copy(A_h.at[rowv.at[j]], bufA.at[b], gs[b])
            pltpu.async_copy(B_h.at[colv.at[j]], bufB.at[b], gs[b])
            pltpu.async_copy(x_h.at[rowv.at[j]], bufxr.at[b], gs[b])
            pltpu.async_copy(x_h.at[colv.at[j]], bufxc.at[b], gs[b])

        def drain_g(j, b):
            pltpu.make_async_copy(A_h.at[rowv.at[j]], bufA.at[b], gs[b]).wait()
            pltpu.make_async_copy(B_h.at[colv.at[j]], bufB.at[b], gs[b]).wait()
            pltpu.make_async_copy(x_h.at[rowv.at[j]], bufxr.at[b], gs[b]).wait()
            pltpu.make_async_copy(x_h.at[colv.at[j]], bufxc.at[b], gs[b]).wait()

        def issue_w(j, b):
            off = (base_chunk + j) * CHUNK
            pltpu.async_copy(bufA.at[b], oT.at[pl.ds(off, CHUNK)], ws[b])
            pltpu.async_copy(bufxr.at[b], oxr.at[pl.ds(off, CHUNK)], ws[b])
            pltpu.async_copy(bufxc.at[b], oxc.at[pl.ds(off, CHUNK)], ws[b])

        def drain_w(j, b):
            off = (base_chunk + j) * CHUNK
            pltpu.make_async_copy(bufA.at[b], oT.at[pl.ds(off, CHUNK)], ws[b]).wait()
            pltpu.make_async_copy(bufxr.at[b], oxr.at[pl.ds(off, CHUNK)], ws[b]).wait()
            pltpu.make_async_copy(bufxc.at[b], oxc.at[pl.ds(off, CHUNK)], ws[b]).wait()

        issue_g(0, 0)

        def outer(jj, carry):
            for b in (0, 1):
                j = jj * 2 + b
                drain_g(j, b)

                @pl.when(j >= 1)
                def _():
                    drain_w(j - 1, 1 - b)

                @pl.when(j + 1 < cpw)
                def _():
                    issue_g(j + 1, 1 - b)

                def add_body(i, carry2):
                    for cc in range(hf // 16):
                        sl = pl.ds(cc * 16, 16)
                        bufA[b, i, sl] = bufA[b, i, sl] + bufB[b, i, sl]
                    return carry2

                lax.fori_loop(0, CHUNK, add_body, 0)

                issue_w(j, b)
            return carry

        lax.fori_loop(0, cpw // 2, outer, 0)
        drain_w(cpw - 1, 1)

    return k(A, B, x4, rowg2d, colg2d)


def _edge_body(t1_r, xr_r, xc_r, ea_r, We1c_r, wr_r, We2_r, be2_r,
               Wc1_r, bc1_r, Wc2_r, bc2_r, out_r):
    d = xr_r[...] - xc_r[...]
    radial = jnp.sum(d * d, axis=1, keepdims=True)
    pre1 = (t1_r[...] + radial * wr_r[...]
            + jnp.dot(ea_r[...], We1c_r[...], preferred_element_type=F32))
    m = _silu(pre1)
    m = _silu(jnp.dot(m, We2_r[...], preferred_element_type=F32) + be2_r[...])
    p = _silu(jnp.dot(m, Wc1_r[...], preferred_element_type=F32) + bc1_r[...])
    cw = jnp.dot(p, Wc2_r[...], preferred_element_type=F32) + bc2_r[...]
    trans = (d * cw)[:, 0:4]
    be = m.shape[0]
    ones = jnp.ones((be, 1), F32)
    zeros = jnp.zeros((be, PW - 69), F32)
    out_r[...] = jnp.concatenate([m, trans, ones, zeros], axis=1)


def _tc_edge(t1, xr4, xc4, ea, We1c, wr, We2, be2, Wc1, bc1, Wc2, bc2, be):
    e_pad, hf = t1.shape
    enf = ea.shape[1]
    grid = (e_pad // be,)
    full = lambda a: pl.BlockSpec(a.shape, lambda i: (0,) * a.ndim)
    return pl.pallas_call(
        _edge_body,
        grid=grid,
        in_specs=[pl.BlockSpec((be, hf), lambda i: (i, 0)),
                  pl.BlockSpec((be, 8), lambda i: (i, 0)),
                  pl.BlockSpec((be, 8), lambda i: (i, 0)),
                  pl.BlockSpec((be, enf), lambda i: (i, 0)),
                  full(We1c), full(wr), full(We2), full(be2),
                  full(Wc1), full(bc1), full(Wc2), full(bc2)],
        out_specs=pl.BlockSpec((be, PW), lambda i: (i, 0)),
        out_shape=jax.ShapeDtypeStruct((e_pad, PW), F32),
    )(t1, xr4, xc4, ea, We1c, wr, We2, be2, Wc1, bc1, Wc2, bc2)


def _sc_scatter(comb, rs2d, zeros, n_acc, cpw):
    rows_per_tile = n_acc // 16
    mesh = plsc.VectorSubcoreMesh(core_axis_name="c", subcore_axis_name="s")

    @functools.partial(
        pl.kernel,
        mesh=mesh,
        out_type=jax.ShapeDtypeStruct((2, n_acc, PW), F32),
        scratch_types=[pltpu.VMEM((cpw, CHUNK), jnp.int32),
                       pltpu.VMEM((2, CHUNK, PW), F32),
                       pltpu.VMEM_SHARED((n_acc, PW), F32),
                       pltpu.SemaphoreType.DMA,
                       pltpu.SemaphoreType.DMA],
        compiler_params=pltpu.CompilerParams(use_tc_tiling_on_sc=False),
    )
    def k(comb_h, rs_h, z_h, out_h, rsv, buf, acc, l0, l1):
        c = lax.axis_index("c")
        s = lax.axis_index("s")
        wid = c * 16 + s
        pltpu.sync_copy(z_h.at[pl.ds(s * rows_per_tile, rows_per_tile)],
                        acc.at[pl.ds(s * rows_per_tile, rows_per_tile)])
        pltpu.sync_copy(rs_h.at[wid], rsv)
        plsc.subcore_barrier()
        ls = (l0, l1)

        def issue_l(j, b):
            off = (wid * cpw + j) * CHUNK
            pltpu.async_copy(comb_h.at[pl.ds(off, CHUNK)], buf.at[b], ls[b])

        def drain_l(j, b):
            off = (wid * cpw + j) * CHUNK
            pltpu.make_async_copy(comb_h.at[pl.ds(off, CHUNK)], buf.at[b],
                                  ls[b]).wait()

        issue_l(0, 0)

        def body(jj, carry):
            for b in (0, 1):
                j = jj * 2 + b
                drain_l(j, b)

                @pl.when(j + 1 < cpw)
                def _():
                    issue_l(j + 1, 1 - b)

                pltpu.sync_copy(buf.at[b], acc.at[rsv.at[j]], add=True)
            return carry

        lax.fori_loop(0, cpw // 2, body, 0)
        plsc.subcore_barrier()
        pltpu.sync_copy(acc.at[pl.ds(s * rows_per_tile, rows_per_tile)],
                        out_h.at[c, pl.ds(s * rows_per_tile, rows_per_tile)])

    return k(comb, rs2d, zeros)


def _node_body(hh_r, x_r, p0_r, p1_r, Wn1a_r, Wn1b_r, bn1_r, Wn2_r, bn2_r,
               Wo_r, bo_r, out_r):
    p0 = p0_r[...]
    p1 = p1_r[...]
    magg = p0[:, 0:64] + p1[:, 0:64]
    tsum = p0[:, 64:67] + p1[:, 64:67]
    cnt = p0[:, 68:69] + p1[:, 68:69]
    x_out = x_r[...] + tsum / jnp.maximum(cnt, 1.0)
    hh = hh_r[...]
    h2 = _silu(jnp.dot(hh, Wn1a_r[...], preferred_element_type=F32)
               + jnp.dot(magg, Wn1b_r[...], preferred_element_type=F32)
               + bn1_r[...])
    h2 = jnp.dot(h2, Wn2_r[...], preferred_element_type=F32) + bn2_r[...]
    emb = jnp.dot(h2, Wo_r[...], preferred_element_type=F32) + bo_r[...]
    out_r[...] = jnp.concatenate([emb, x_out], axis=1)


def _tc_node(hh, x, p0, p1, Wn1a, Wn1b, bn1, Wn2, bn2, W_out, b_out, bn):
    n, hf = hh.shape
    emb_nf = W_out.shape[1]
    grid = (n // bn,)
    full = lambda a: pl.BlockSpec(a.shape, lambda i: (0,) * a.ndim)
    return pl.pallas_call(
        _node_body,
        grid=grid,
        in_specs=[pl.BlockSpec((bn, hf), lambda i: (i, 0)),
                  pl.BlockSpec((bn, 3), lambda i: (i, 0)),
                  pl.BlockSpec((bn, PW), lambda i: (i, 0)),
                  pl.BlockSpec((bn, PW), lambda i: (i, 0)),
                  full(Wn1a), full(Wn1b), full(bn1), full(Wn2), full(bn2),
                  full(W_out), full(b_out)],
        out_specs=pl.BlockSpec((bn, emb_nf + 3), lambda i: (i, 0)),
        out_shape=jax.ShapeDtypeStruct((n, emb_nf + 3), F32),
    )(hh, x, p0, p1, Wn1a, Wn1b, bn1, Wn2, bn2, W_out, b_out)


def kernel(h, x, edge_index, edge_attr, W_in, b_in, We1, be1, We2, be2,
           Wc1, bc1, Wc2, bc2, Wn1, bn1, Wn2, bn2, W_out, b_out):
    n = h.shape[0]
    e = edge_index.shape[1]
    hf = W_in.shape[1]

    cpw = -(-e // (NW * CHUNK))
    cpw = cpw + (cpw % 2)
    e_pad = NW * cpw * CHUNK
    pad = e_pad - e
    n_acc = ((n + 1 + 127) // 128) * 128

    row = edge_index[0].astype(jnp.int32)
    col = edge_index[1].astype(jnp.int32)
    rowg2d = jnp.concatenate([row, jnp.zeros((pad,), jnp.int32)]).reshape(NW, cpw, CHUNK)
    colg2d = jnp.concatenate([col, jnp.zeros((pad,), jnp.int32)]).reshape(NW, cpw, CHUNK)
    rs2d = jnp.concatenate([row, jnp.full((pad,), n, jnp.int32)]).reshape(NW, cpw, CHUNK)
    ea_pad = jnp.pad(edge_attr, ((0, pad), (0, 0)))
    x4 = jnp.pad(x, ((0, 0), (0, 5)))

    We1a = We1[:hf]
    We1b = We1[hf:2 * hf]
    wr = We1[2 * hf:2 * hf + 1]
    We1c = We1[2 * hf + 1:]
    b2 = lambda v: v.reshape(1, -1)

    hh, A, B = _tc_pre(h, W_in, b2(b_in), We1a, We1b, b2(be1), bn=1000)
    t1, xr4, xc4 = _sc_gather(A, B, x4, rowg2d, colg2d, e_pad, cpw)
    comb = _tc_edge(t1, xr4, xc4, ea_pad, We1c, wr, We2, b2(be2),
                    Wc1, b2(bc1), Wc2, b2(bc2), be=2048)
    zeros = jnp.zeros((n_acc, PW), F32)
    partials = _sc_scatter(comb, rs2d, zeros, n_acc, cpw)
    p0 = partials[0, :n]
    p1 = partials[1, :n]
    out = _tc_node(hh, x, p0, p1, Wn1[:hf], Wn1[hf:], b2(bn1), Wn2, b2(bn2),
                   W_out, b2(b_out), bn=1000)
    return out


# speedup vs baseline: 3.8046x; 1.0875x over previous
import functools

import jax
import jax.numpy as jnp
from jax import lax
from jax.experimental import pallas as pl
from jax.experimental.pallas import tpu as pltpu
from jax.experimental.pallas import tpu_sc as plsc

F32 = jnp.float32
NW = 32
CHUNK = 128
PW = 72


def _silu(v):
    return v * jax.nn.sigmoid(v)


def _pre_body(h_r, Win_r, bin_r, We1a_r, We1b_r, be1_r, hh_r, A_r, B_r):
    hh = jnp.dot(h_r[...], Win_r[...], preferred_element_type=F32) + bin_r[...]
    hh_r[...] = hh
    A_r[...] = jnp.dot(hh, We1a_r[...], preferred_element_type=F32) + be1_r[...]
    B_r[...] = jnp.dot(hh, We1b_r[...], preferred_element_type=F32)


def _tc_pre(h, W_in, b_in, We1a, We1b, be1, bn):
    n, in_nf = h.shape
    hf = W_in.shape[1]
    grid = (n // bn,)
    full = lambda a: pl.BlockSpec(a.shape, lambda i: (0,) * a.ndim)
    return pl.pallas_call(
        _pre_body,
        grid=grid,
        in_specs=[pl.BlockSpec((bn, in_nf), lambda i: (i, 0)),
                  full(W_in), full(b_in), full(We1a), full(We1b), full(be1)],
        out_specs=[pl.BlockSpec((bn, hf), lambda i: (i, 0))] * 3,
        out_shape=[jax.ShapeDtypeStruct((n, hf), F32)] * 3,
    )(h, W_in, b_in, We1a, We1b, be1)


def _sc_gather(A, B, x4, rowg2d, colg2d, e_pad, cpw):
    hf = A.shape[1]
    mesh = plsc.VectorSubcoreMesh(core_axis_name="c", subcore_axis_name="s")

    @functools.partial(
        pl.kernel,
        mesh=mesh,
        out_type=(jax.ShapeDtypeStruct((e_pad, hf), F32),
                  jax.ShapeDtypeStruct((e_pad, 8), F32),
                  jax.ShapeDtypeStruct((e_pad, 8), F32)),
        scratch_types=[pltpu.VMEM((cpw, CHUNK), jnp.int32),
                       pltpu.VMEM((cpw, CHUNK), jnp.int32),
                       pltpu.VMEM((2, CHUNK, hf), F32),
                       pltpu.VMEM((2, CHUNK, hf), F32),
                       pltpu.VMEM((2, CHUNK, 8), F32),
                       pltpu.VMEM((2, CHUNK, 8), F32),
                       pltpu.SemaphoreType.DMA,
                       pltpu.SemaphoreType.DMA,
                       pltpu.SemaphoreType.DMA,
                       pltpu.SemaphoreType.DMA],
        compiler_params=pltpu.CompilerParams(use_tc_tiling_on_sc=False),
    )
    def k(A_h, B_h, x_h, rg_h, cg_h, oT, oxr, oxc,
          rowv, colv, bufA, bufB, bufxr, bufxc, g0, g1, w0, w1):
        c = lax.axis_index("c")
        s = lax.axis_index("s")
        wid = c * 16 + s
        base_chunk = wid * cpw
        pltpu.sync_copy(rg_h.at[wid], rowv)
        pltpu.sync_copy(cg_h.at[wid], colv)
        gs = (g0, g1)
        ws = (w0, w1)

        def issue_g(j, b):
            pltpu.async_copy(A_h.at[rowv.at[j]], bufA.at[b], gs[b])
            pltpu.async_copy(B_h.at[colv.at[j]], bufB.at[b], gs[b])
            pltpu.async_copy(x_h.at[rowv.at[j]], bufxr.at[b], gs[b])
            pltpu.async_copy(x_h.at[colv.at[j]], bufxc.at[b], gs[b])

        def drain_g(j, b):
            pltpu.make_async_copy(A_h.at[rowv.at[j]], bufA.at[b], gs[b]).wait()
            pltpu.make_async_copy(B_h.at[colv.at[j]], bufB.at[b], gs[b]).wait()
            pltpu.make_async_copy(x_h.at[rowv.at[j]], bufxr.at[b], gs[b]).wait()
            pltpu.make_async_copy(x_h.at[colv.at[j]], bufxc.at[b], gs[b]).wait()

        def issue_w(j, b):
            off = (base_chunk + j) * CHUNK
            pltpu.async_copy(bufA.at[b], oT.at[pl.ds(off, CHUNK)], ws[b])
            pltpu.async_copy(bufxr.at[b], oxr.at[pl.ds(off, CHUNK)], ws[b])
            pltpu.async_copy(bufxc.at[b], oxc.at[pl.ds(off, CHUNK)], ws[b])

        def drain_w(j, b):
            off = (base_chunk + j) * CHUNK
            pltpu.make_async_copy(bufA.at[b], oT.at[pl.ds(off, CHUNK)], ws[b]).wait()
            pltpu.make_async_copy(bufxr.at[b], oxr.at[pl.ds(off, CHUNK)], ws[b]).wait()
            pltpu.make_async_copy(bufxc.at[b], oxc.at[pl.ds(off, CHUNK)], ws[b]).wait()

        issue_g(0, 0)

        def outer(jj, carry):
            for b in (0, 1):
                j = jj * 2 + b
                drain_g(j, b)

                @pl.when(j >= 1)
                def _():
                    drain_w(j - 1, 1 - b)

                @pl.when(j + 1 < cpw)
                def _():
                    issue_g(j + 1, 1 - b)

                def add_body(i, carry2):
                    for cc in range(hf // 16):
                        sl = pl.ds(cc * 16, 16)
                        bufA[b, i, sl] = bufA[b, i, sl] + bufB[b, i, sl]
                    return carry2

                lax.fori_loop(0, CHUNK, add_body, 0)

                issue_w(j, b)
            return carry

        lax.fori_loop(0, cpw // 2, outer, 0)
        drain_w(cpw - 1, 1)

    return k(A, B, x4, rowg2d, colg2d)


def _edge_body(t1_r, xr_r, xc_r, ea_r, We1c_r, wr_r, We2_r, be2_r,
               Wc1_r, bc1_r, Wc2_r, bc2_r, out_r):
    d = xr_r[...] - xc_r[...]
    radial = jnp.sum(d * d, axis=1, keepdims=True)
    pre1 = (t1_r[...] + radial * wr_r[...]
            + jnp.dot(ea_r[...], We1c_r[...], preferred_element_type=F32))
    m = _silu(pre1)
    m = _silu(jnp.dot(m, We2_r[...], preferred_element_type=F32) + be2_r[...])
    p = _silu(jnp.dot(m, Wc1_r[...], preferred_element_type=F32) + bc1_r[...])
    cw = jnp.dot(p, Wc2_r[...], preferred_element_type=F32) + bc2_r[...]
    trans = (d * cw)[:, 0:4]
    be = m.shape[0]
    ones = jnp.ones((be, 1), F32)
    zeros = jnp.zeros((be, PW - 69), F32)
    out_r[...] = jnp.concatenate([m, trans, ones, zeros], axis=1)


def _tc_edge(t1, xr4, xc4, ea, We1c, wr, We2, be2, Wc1, bc1, Wc2, bc2, be):
    e_pad, hf = t1.shape
    enf = ea.shape[1]
    grid = (e_pad // be,)
    full = lambda a: pl.BlockSpec(a.shape, lambda i: (0,) * a.ndim)
    return pl.pallas_call(
        _edge_body,
        grid=grid,
        in_specs=[pl.BlockSpec((be, hf), lambda i: (i, 0)),
                  pl.BlockSpec((be, 8), lambda i: (i, 0)),
                  pl.BlockSpec((be, 8), lambda i: (i, 0)),
                  pl.BlockSpec((be, enf), lambda i: (i, 0)),
                  full(We1c), full(wr), full(We2), full(be2),
                  full(Wc1), full(bc1), full(Wc2), full(bc2)],
        out_specs=pl.BlockSpec((be, PW), lambda i: (i, 0)),
        out_shape=jax.ShapeDtypeStruct((e_pad, PW), F32),
    )(t1, xr4, xc4, ea, We1c, wr, We2, be2, Wc1, bc1, Wc2, bc2)


def _sc_scatter(comb, rs2d, zeros, n_acc, cpw):
    rows_per_tile = n_acc // 16
    mesh = plsc.VectorSubcoreMesh(core_axis_name="c", subcore_axis_name="s")

    @functools.partial(
        pl.kernel,
        mesh=mesh,
        out_type=jax.ShapeDtypeStruct((2, n_acc, PW), F32),
        scratch_types=[pltpu.VMEM((cpw, CHUNK), jnp.int32),
                       pltpu.VMEM((2, CHUNK, PW), F32),
                       pltpu.VMEM_SHARED((n_acc, PW), F32),
                       pltpu.SemaphoreType.DMA,
                       pltpu.SemaphoreType.DMA],
        compiler_params=pltpu.CompilerParams(use_tc_tiling_on_sc=False),
    )
    def k(comb_h, rs_h, z_h, out_h, rsv, buf, acc, l0, l1):
        c = lax.axis_index("c")
        s = lax.axis_index("s")
        wid = c * 16 + s
        pltpu.sync_copy(z_h.at[pl.ds(s * rows_per_tile, rows_per_tile)],
                        acc.at[pl.ds(s * rows_per_tile, rows_per_tile)])
        pltpu.sync_copy(rs_h.at[wid], rsv)
        plsc.subcore_barrier()
        ls = (l0, l1)

        def issue_l(j, b):
            off = (wid * cpw + j) * CHUNK
            pltpu.async_copy(comb_h.at[pl.ds(off, CHUNK)], buf.at[b], ls[b])

        def drain_l(j, b):
            off = (wid * cpw + j) * CHUNK
            pltpu.make_async_copy(comb_h.at[pl.ds(off, CHUNK)], buf.at[b],
                                  ls[b]).wait()

        issue_l(0, 0)

        def body(jj, carry):
            for b in (0, 1):
                j = jj * 2 + b
                drain_l(j, b)

                @pl.when(j + 1 < cpw)
                def _():
                    issue_l(j + 1, 1 - b)

                pltpu.sync_copy(buf.at[b], acc.at[rsv.at[j]], add=True)
            return carry

        lax.fori_loop(0, cpw // 2, body, 0)
        plsc.subcore_barrier()
        pltpu.sync_copy(acc.at[pl.ds(s * rows_per_tile, rows_per_tile)],
                        out_h.at[c, pl.ds(s * rows_per_tile, rows_per_tile)])

    return k(comb, rs2d, zeros)


def _node_body(hh_r, x_r, *rest):
    (Wn1a_r, Wn1b_r, bn1_r, Wn2_r, bn2_r, Wo_r, bo_r, out_r) = rest[-8:]
    p_refs = rest[:-8]
    p = p_refs[0][...]
    for pr in p_refs[1:]:
        p = p + pr[...]
    magg = p[:, 0:64]
    tsum = p[:, 64:67]
    cnt = p[:, 68:69]
    x_out = x_r[...] + tsum / jnp.maximum(cnt, 1.0)
    hh = hh_r[...]
    h2 = _silu(jnp.dot(hh, Wn1a_r[...], preferred_element_type=F32)
               + jnp.dot(magg, Wn1b_r[...], preferred_element_type=F32)
               + bn1_r[...])
    h2 = jnp.dot(h2, Wn2_r[...], preferred_element_type=F32) + bn2_r[...]
    emb = jnp.dot(h2, Wo_r[...], preferred_element_type=F32) + bo_r[...]
    out_r[...] = jnp.concatenate([emb, x_out], axis=1)


def _tc_node(hh, x, ps, Wn1a, Wn1b, bn1, Wn2, bn2, W_out, b_out, bn):
    n, hf = hh.shape
    emb_nf = W_out.shape[1]
    grid = (n // bn,)
    full = lambda a: pl.BlockSpec(a.shape, lambda i: (0,) * a.ndim)
    return pl.pallas_call(
        _node_body,
        grid=grid,
        in_specs=[pl.BlockSpec((bn, hf), lambda i: (i, 0)),
                  pl.BlockSpec((bn, 3), lambda i: (i, 0))]
                 + [pl.BlockSpec((bn, PW), lambda i: (i, 0))] * len(ps)
                 + [full(Wn1a), full(Wn1b), full(bn1), full(Wn2), full(bn2),
                    full(W_out), full(b_out)],
        out_specs=pl.BlockSpec((bn, emb_nf + 3), lambda i: (i, 0)),
        out_shape=jax.ShapeDtypeStruct((n, emb_nf + 3), F32),
    )(hh, x, *ps, Wn1a, Wn1b, bn1, Wn2, bn2, W_out, b_out)


def kernel(h, x, edge_index, edge_attr, W_in, b_in, We1, be1, We2, be2,
           Wc1, bc1, Wc2, bc2, Wn1, bn1, Wn2, bn2, W_out, b_out):
    n = h.shape[0]
    e = edge_index.shape[1]
    hf = W_in.shape[1]

    nseg = 4
    cps = -(-e // (nseg * NW * CHUNK))
    cps = cps + (cps % 2)
    e_seg = NW * cps * CHUNK
    e_pad = nseg * e_seg
    pad = e_pad - e
    n_acc = ((n + 1 + 127) // 128) * 128

    row = edge_index[0].astype(jnp.int32)
    col = edge_index[1].astype(jnp.int32)
    rowg = jnp.concatenate([row, jnp.zeros((pad,), jnp.int32)])
    colg = jnp.concatenate([col, jnp.zeros((pad,), jnp.int32)])
    rs = jnp.concatenate([row, jnp.full((pad,), n, jnp.int32)])
    rowg4 = rowg.reshape(nseg, NW, cps, CHUNK)
    colg4 = colg.reshape(nseg, NW, cps, CHUNK)
    rs4 = rs.reshape(nseg, NW, cps, CHUNK)
    ea_pad = jnp.pad(edge_attr, ((0, pad), (0, 0)))
    x4 = jnp.pad(x, ((0, 0), (0, 5)))

    We1a = We1[:hf]
    We1b = We1[hf:2 * hf]
    wr = We1[2 * hf:2 * hf + 1]
    We1c = We1[2 * hf + 1:]
    b2 = lambda v: v.reshape(1, -1)

    hh, A, B = _tc_pre(h, W_in, b2(b_in), We1a, We1b, b2(be1), bn=1000)
    zeros = jnp.zeros((n_acc, PW), F32)
    ps = []
    for sgi in range(nseg):
        t1, xr4, xc4 = _sc_gather(A, B, x4, rowg4[sgi], colg4[sgi], e_seg, cps)
        comb = _tc_edge(t1, xr4, xc4,
                        lax.dynamic_slice_in_dim(ea_pad, sgi * e_seg, e_seg),
                        We1c, wr, We2, b2(be2), Wc1, b2(bc1), Wc2, b2(bc2),
                        be=2048)
        partials = _sc_scatter(comb, rs4[sgi], zeros, n_acc, cps)
        ps.append(partials[0, :n])
        ps.append(partials[1, :n])
    out = _tc_node(hh, x, ps, Wn1[:hf], Wn1[hf:], b2(bn1), Wn2, b2(bn2),
                   W_out, b2(b_out), bn=1000)
    return out


# speedup vs baseline: 4.5224x; 1.1887x over previous
import functools

import jax
import jax.numpy as jnp
from jax import lax
from jax.experimental import pallas as pl
from jax.experimental.pallas import tpu as pltpu
from jax.experimental.pallas import tpu_sc as plsc

F32 = jnp.float32
NW = 32
CHUNK = 128
PW = 72


def _silu(v):
    return v * jax.nn.sigmoid(v)


GW = 80


def _pre_body(h_r, x8_r, Win_r, bin_r, We1a_r, We1b_r, be1_r,
              hh_r, Ax_r, Bx_r):
    hh = jnp.dot(h_r[...], Win_r[...], preferred_element_type=F32) + bin_r[...]
    hh_r[...] = hh
    A = jnp.dot(hh, We1a_r[...], preferred_element_type=F32) + be1_r[...]
    B = jnp.dot(hh, We1b_r[...], preferred_element_type=F32)
    x8 = x8_r[...]
    z = jnp.zeros((hh.shape[0], GW - 72), F32)
    Ax_r[...] = jnp.concatenate([A, x8, z], axis=1)
    Bx_r[...] = jnp.concatenate([B, x8, z], axis=1)


def _tc_pre(h, x8, W_in, b_in, We1a, We1b, be1, bn):
    n, in_nf = h.shape
    hf = W_in.shape[1]
    grid = (n // bn,)
    full = lambda a: pl.BlockSpec(a.shape, lambda i: (0,) * a.ndim)
    return pl.pallas_call(
        _pre_body,
        grid=grid,
        in_specs=[pl.BlockSpec((bn, in_nf), lambda i: (i, 0)),
                  pl.BlockSpec((bn, 8), lambda i: (i, 0)),
                  full(W_in), full(b_in), full(We1a), full(We1b), full(be1)],
        out_specs=[pl.BlockSpec((bn, hf), lambda i: (i, 0)),
                   pl.BlockSpec((bn, GW), lambda i: (i, 0)),
                   pl.BlockSpec((bn, GW), lambda i: (i, 0))],
        out_shape=[jax.ShapeDtypeStruct((n, hf), F32),
                   jax.ShapeDtypeStruct((n, GW), F32),
                   jax.ShapeDtypeStruct((n, GW), F32)],
    )(h, x8, W_in, b_in, We1a, We1b, be1)


def _sc_gather(Ax, Bx, rowg2d, colg2d, e_pad, cpw):
    mesh = plsc.VectorSubcoreMesh(core_axis_name="c", subcore_axis_name="s")

    @functools.partial(
        pl.kernel,
        mesh=mesh,
        out_type=jax.ShapeDtypeStruct((e_pad, GW), F32),
        scratch_types=[pltpu.VMEM((cpw, CHUNK), jnp.int32),
                       pltpu.VMEM((cpw, CHUNK), jnp.int32),
                       pltpu.VMEM((4, CHUNK, GW), F32),
                       pltpu.VMEM((4, CHUNK, GW), F32),
                       pltpu.SemaphoreType.DMA,
                       pltpu.SemaphoreType.DMA,
                       pltpu.SemaphoreType.DMA,
                       pltpu.SemaphoreType.DMA,
                       pltpu.SemaphoreType.DMA,
                       pltpu.SemaphoreType.DMA,
                       pltpu.SemaphoreType.DMA,
                       pltpu.SemaphoreType.DMA],
        compiler_params=pltpu.CompilerParams(use_tc_tiling_on_sc=False),
    )
    def k(A_h, B_h, rg_h, cg_h, oG,
          rowv, colv, bufA, bufB,
          g0, g1, g2, g3, w0, w1, w2, w3):
        c = lax.axis_index("c")
        s = lax.axis_index("s")
        wid = c * 16 + s
        base_chunk = wid * cpw
        pltpu.sync_copy(rg_h.at[wid], rowv)
        pltpu.sync_copy(cg_h.at[wid], colv)
        gs = (g0, g1, g2, g3)
        ws = (w0, w1, w2, w3)

        def issue_g(j, b):
            pltpu.async_copy(A_h.at[rowv.at[j]], bufA.at[b], gs[b])
            pltpu.async_copy(B_h.at[colv.at[j]], bufB.at[b], gs[b])

        def drain_g(j, b):
            pltpu.make_async_copy(A_h.at[rowv.at[j]], bufA.at[b], gs[b]).wait()
            pltpu.make_async_copy(B_h.at[colv.at[j]], bufB.at[b], gs[b]).wait()

        def issue_w(j, b):
            off = (base_chunk + j) * CHUNK
            pltpu.async_copy(bufA.at[b], oG.at[pl.ds(off, CHUNK)], ws[b])

        def drain_w(j, b):
            off = (base_chunk + j) * CHUNK
            pltpu.make_async_copy(bufA.at[b], oG.at[pl.ds(off, CHUNK)],
                                  ws[b]).wait()

        issue_g(0, 0)
        issue_g(1, 1)

        def outer(jj, carry):
            for b in (0, 1, 2, 3):
                j = jj * 4 + b
                drain_g(j, b)

                @pl.when(j >= 2)
                def _():
                    drain_w(j - 2, (b + 2) % 4)

                @pl.when(j + 2 < cpw)
                def _():
                    issue_g(j + 2, (b + 2) % 4)

                def add_body(i, carry2):
                    for cc in range(4):
                        sl = pl.ds(cc * 16, 16)
                        bufA[b, i, sl] = bufA[b, i, sl] + bufB[b, i, sl]
                    sl = pl.ds(64, 16)
                    bufA[b, i, sl] = bufA[b, i, sl] - bufB[b, i, sl]
                    return carry2

                lax.fori_loop(0, CHUNK, add_body, 0)

                issue_w(j, b)
            return carry

        lax.fori_loop(0, cpw // 4, outer, 0)
        drain_w(cpw - 2, (cpw - 2) % 4)
        drain_w(cpw - 1, (cpw - 1) % 4)

    return k(Ax, Bx, rowg2d, colg2d)


def _edge_body(g_r, ea_r, We1c_r, wr_r, We2_r, be2_r,
               Wc1_r, bc1_r, Wc2_r, bc2_r, out_r):
    g = g_r[...]
    t1 = g[:, 0:64]
    d = g[:, 64:72]
    radial = jnp.sum(d * d, axis=1, keepdims=True)
    pre1 = (t1 + radial * wr_r[...]
            + jnp.dot(ea_r[...], We1c_r[...], preferred_element_type=F32))
    m = _silu(pre1)
    m = _silu(jnp.dot(m, We2_r[...], preferred_element_type=F32) + be2_r[...])
    p = _silu(jnp.dot(m, Wc1_r[...], preferred_element_type=F32) + bc1_r[...])
    cw = jnp.dot(p, Wc2_r[...], preferred_element_type=F32) + bc2_r[...]
    trans = (d * cw)[:, 0:4]
    be = m.shape[0]
    ones = jnp.ones((be, 1), F32)
    zeros = jnp.zeros((be, PW - 69), F32)
    out_r[...] = jnp.concatenate([m, trans, ones, zeros], axis=1)


def _tc_edge(g, ea, We1c, wr, We2, be2, Wc1, bc1, Wc2, bc2, be):
    e_pad = g.shape[0]
    enf = ea.shape[1]
    grid = (e_pad // be,)
    full = lambda a: pl.BlockSpec(a.shape, lambda i: (0,) * a.ndim)
    return pl.pallas_call(
        _edge_body,
        grid=grid,
        in_specs=[pl.BlockSpec((be, GW), lambda i: (i, 0)),
                  pl.BlockSpec((be, enf), lambda i: (i, 0)),
                  full(We1c), full(wr), full(We2), full(be2),
                  full(Wc1), full(bc1), full(Wc2), full(bc2)],
        out_specs=pl.BlockSpec((be, PW), lambda i: (i, 0)),
        out_shape=jax.ShapeDtypeStruct((e_pad, PW), F32),
    )(g, ea, We1c, wr, We2, be2, Wc1, bc1, Wc2, bc2)


def _sc_scatter(comb, rs2d, zeros, n_acc, cpw):
    rows_per_tile = n_acc // 16
    mesh = plsc.VectorSubcoreMesh(core_axis_name="c", subcore_axis_name="s")

    @functools.partial(
        pl.kernel,
        mesh=mesh,
        out_type=jax.ShapeDtypeStruct((2, n_acc, PW), F32),
        scratch_types=[pltpu.VMEM((cpw, CHUNK), jnp.int32),
                       pltpu.VMEM((2, CHUNK, PW), F32),
                       pltpu.VMEM_SHARED((n_acc, PW), F32),
                       pltpu.SemaphoreType.DMA,
                       pltpu.SemaphoreType.DMA],
        compiler_params=pltpu.CompilerParams(use_tc_tiling_on_sc=False),
    )
    def k(comb_h, rs_h, z_h, out_h, rsv, buf, acc, l0, l1):
        c = lax.axis_index("c")
        s = lax.axis_index("s")
        wid = c * 16 + s
        pltpu.sync_copy(z_h.at[pl.ds(s * rows_per_tile, rows_per_tile)],
                        acc.at[pl.ds(s * rows_per_tile, rows_per_tile)])
        pltpu.sync_copy(rs_h.at[wid], rsv)
        plsc.subcore_barrier()
        ls = (l0, l1)

        def issue_l(j, b):
            off = (wid * cpw + j) * CHUNK
            pltpu.async_copy(comb_h.at[pl.ds(off, CHUNK)], buf.at[b], ls[b])

        def drain_l(j, b):
            off = (wid * cpw + j) * CHUNK
            pltpu.make_async_copy(comb_h.at[pl.ds(off, CHUNK)], buf.at[b],
                                  ls[b]).wait()

        issue_l(0, 0)

        def body(jj, carry):
            for b in (0, 1):
                j = jj * 2 + b
                drain_l(j, b)

                @pl.when(j + 1 < cpw)
                def _():
                    issue_l(j + 1, 1 - b)

                pltpu.sync_copy(buf.at[b], acc.at[rsv.at[j]], add=True)
            return carry

        lax.fori_loop(0, cpw // 2, body, 0)
        plsc.subcore_barrier()
        pltpu.sync_copy(acc.at[pl.ds(s * rows_per_tile, rows_per_tile)],
                        out_h.at[c, pl.ds(s * rows_per_tile, rows_per_tile)])

    return k(comb, rs2d, zeros)


def _node_body(hh_r, x_r, *rest):
    (Wn1a_r, Wn1b_r, bn1_r, Wn2_r, bn2_r, Wo_r, bo_r, out_r) = rest[-8:]
    p_refs = rest[:-8]
    p = p_refs[0][...]
    for pr in p_refs[1:]:
        p = p + pr[...]
    magg = p[:, 0:64]
    tsum = p[:, 64:67]
    cnt = p[:, 68:69]
    x_out = x_r[...] + tsum / jnp.maximum(cnt, 1.0)
    hh = hh_r[...]
    h2 = _silu(jnp.dot(hh, Wn1a_r[...], preferred_element_type=F32)
               + jnp.dot(magg, Wn1b_r[...], preferred_element_type=F32)
               + bn1_r[...])
    h2 = jnp.dot(h2, Wn2_r[...], preferred_element_type=F32) + bn2_r[...]
    emb = jnp.dot(h2, Wo_r[...], preferred_element_type=F32) + bo_r[...]
    out_r[...] = jnp.concatenate([emb, x_out], axis=1)


def _tc_node(hh, x, ps, Wn1a, Wn1b, bn1, Wn2, bn2, W_out, b_out, bn):
    n, hf = hh.shape
    emb_nf = W_out.shape[1]
    grid = (n // bn,)
    full = lambda a: pl.BlockSpec(a.shape, lambda i: (0,) * a.ndim)
    return pl.pallas_call(
        _node_body,
        grid=grid,
        in_specs=[pl.BlockSpec((bn, hf), lambda i: (i, 0)),
                  pl.BlockSpec((bn, 3), lambda i: (i, 0))]
                 + [pl.BlockSpec((bn, PW), lambda i: (i, 0))] * len(ps)
                 + [full(Wn1a), full(Wn1b), full(bn1), full(Wn2), full(bn2),
                    full(W_out), full(b_out)],
        out_specs=pl.BlockSpec((bn, emb_nf + 3), lambda i: (i, 0)),
        out_shape=jax.ShapeDtypeStruct((n, emb_nf + 3), F32),
    )(hh, x, *ps, Wn1a, Wn1b, bn1, Wn2, bn2, W_out, b_out)


def kernel(h, x, edge_index, edge_attr, W_in, b_in, We1, be1, We2, be2,
           Wc1, bc1, Wc2, bc2, Wn1, bn1, Wn2, bn2, W_out, b_out):
    n = h.shape[0]
    e = edge_index.shape[1]
    hf = W_in.shape[1]

    nseg = 4
    cps = -(-e // (nseg * NW * CHUNK))
    cps = ((cps + 3) // 4) * 4
    e_seg = NW * cps * CHUNK
    e_pad = nseg * e_seg
    pad = e_pad - e
    n_acc = ((n + 1 + 127) // 128) * 128

    row = edge_index[0].astype(jnp.int32)
    col = edge_index[1].astype(jnp.int32)
    rowg = jnp.concatenate([row, jnp.zeros((pad,), jnp.int32)])
    colg = jnp.concatenate([col, jnp.zeros((pad,), jnp.int32)])
    rs = jnp.concatenate([row, jnp.full((pad,), n, jnp.int32)])
    rowg4 = rowg.reshape(nseg, NW, cps, CHUNK)
    colg4 = colg.reshape(nseg, NW, cps, CHUNK)
    rs4 = rs.reshape(nseg, NW, cps, CHUNK)
    ea_pad = jnp.pad(edge_attr, ((0, pad), (0, 0)))
    x4 = jnp.pad(x, ((0, 0), (0, 5)))

    We1a = We1[:hf]
    We1b = We1[hf:2 * hf]
    wr = We1[2 * hf:2 * hf + 1]
    We1c = We1[2 * hf + 1:]
    b2 = lambda v: v.reshape(1, -1)

    hh, Ax, Bx = _tc_pre(h, x4, W_in, b2(b_in), We1a, We1b, b2(be1), bn=1000)
    zeros = jnp.zeros((n_acc, PW), F32)
    ps = []
    for sgi in range(nseg):
        g = _sc_gather(Ax, Bx, rowg4[sgi], colg4[sgi], e_seg, cps)
        comb = _tc_edge(g,
                        lax.dynamic_slice_in_dim(ea_pad, sgi * e_seg, e_seg),
                        We1c, wr, We2, b2(be2), Wc1, b2(bc1), Wc2, b2(bc2),
                        be=2048)
        partials = _sc_scatter(comb, rs4[sgi], zeros, n_acc, cps)
        ps.append(partials[0, :n])
        ps.append(partials[1, :n])
    out = _tc_node(hh, x, ps, Wn1[:hf], Wn1[hf:], b2(bn1), Wn2, b2(bn2),
                   W_out, b2(b_out), bn=1000)
    return out
